# Initial kernel scaffold; baseline (speedup 1.0000x reference)
import jax
import jax.numpy as jnp
from jax.experimental import pallas as pl


def kernel(x, table):
    raise NotImplementedError("write your pallas kernel here")


# speedup vs baseline: 1.8917x; 1.8917x over previous
import functools

import jax
import jax.numpy as jnp
from jax import lax
from jax.experimental import pallas as pl
from jax.experimental.pallas import tpu as pltpu
from jax.experimental.pallas import tpu_sc as plsc

D = 64
LANES = 16
NCORES = 2
NSUB = 16
NW = NCORES * NSUB
CHUNK = 128
NBUF = 2


_GDN = lax.GatherDimensionNumbers(
    offset_dims=(), collapsed_slice_dims=(0,), start_index_map=(0,))


def _shuffle16(s, idx):
    return lax.gather(
        s, idx.reshape(LANES, 1), _GDN, slice_sizes=(1,),
        mode=lax.GatherScatterMode.PROMISE_IN_BOUNDS)


def _lane_sum16(s):
    iota = jnp.arange(LANES, dtype=jnp.int32)
    for k in (8, 4, 2, 1):
        s = s + _shuffle16(s, iota ^ k)
    return s


def _rsqrt16(a):
    i = lax.bitcast_convert_type(a, jnp.int32)
    y = lax.bitcast_convert_type(
        jnp.int32(0x5F3759DF) - lax.shift_right_logical(i, 1), jnp.float32)
    for _ in range(3):
        y = y * (1.5 - 0.5 * a * y * y)
    return y


def _make_sc_kernel(nchunk):
    mesh = plsc.VectorSubcoreMesh(core_axis_name="c", subcore_axis_name="s")

    @functools.partial(
        pl.kernel,
        mesh=mesh,
        compiler_params=pltpu.CompilerParams(use_tc_tiling_on_sc=False),
        out_type=jax.ShapeDtypeStruct((NW, nchunk, CHUNK, D), jnp.float32),
        scratch_types=[
            pltpu.VMEM((nchunk, CHUNK), jnp.int32),
            pltpu.VMEM((NBUF, CHUNK, D), jnp.float32),
            pltpu.VMEM((NBUF, CHUNK, D), jnp.float32),
            pltpu.SemaphoreType.DMA,
            pltpu.SemaphoreType.DMA,
            pltpu.SemaphoreType.DMA,
            pltpu.SemaphoreType.DMA,
        ],
    )
    def k(x_hbm, table_hbm, out_hbm, idx_v, in_v, out_v, g0, g1, s0, s1):
        gsem = [g0, g1]
        ssem = [s0, s1]
        wid = lax.axis_index("s") * NCORES + lax.axis_index("c")
        pltpu.sync_copy(x_hbm.at[wid], idx_v)

        def gather(c, b):
            return pltpu.make_async_copy(
                table_hbm.at[idx_v.at[c]], in_v.at[b], gsem[b])

        def scatter(c, b):
            return pltpu.make_async_copy(
                out_v.at[b], out_hbm.at[wid, c], ssem[b])

        for b in range(NBUF):
            gather(b, b).start()

        def row_body(r, carry, inb, outb):
            v = [inb[r, pl.ds(LANES * j, LANES)] for j in range(D // LANES)]
            s = v[0] * v[0]
            for j in range(1, D // LANES):
                s = s + v[j] * v[j]
            inv = _rsqrt16(_lane_sum16(s))
            for j in range(D // LANES):
                outb[r, pl.ds(LANES * j, LANES)] = v[j] * inv
            return carry

        def outer(g, carry):
            for b in range(NBUF):
                c = g * NBUF + b
                gather(c, b).wait()

                @pl.when(g >= 1)
                def _():
                    scatter(c - NBUF, b).wait()

                lax.fori_loop(
                    0, CHUNK,
                    functools.partial(row_body, inb=in_v.at[b], outb=out_v.at[b]),
                    0)
                scatter(c, b).start()

                @pl.when(g < nchunk // NBUF - 1)
                def _():
                    gather(c + NBUF, b).start()
            return carry

        lax.fori_loop(0, nchunk // NBUF, outer, 0)
        for b in range(NBUF):
            scatter(nchunk - NBUF + b, b).wait()

    return k


def kernel(x, table):
    B, H = x.shape
    total = B * H
    nchunk = total // (NW * CHUNK)
    xw = x.astype(jnp.int32).reshape(NW, nchunk, CHUNK)
    out = _make_sc_kernel(nchunk)(xw, table)
    return out.reshape(B, H, D)
